# baseline (speedup 1.0000x reference)
import jax
import jax.numpy as jnp
from jax import lax
from jax.experimental import pallas as pl
from jax.experimental.pallas import tpu as pltpu
from jax.experimental.pallas import tpu_sc as plsc

_NUM_FIXED = 256
_TOTAL_VT = 1024
_TOKEN_DIM = 4096
_BATCH = 16

_NC = 2
_NS = 16
_NW = _NC * _NS

_B_FLAT = _BATCH * _TOTAL_VT
_B_PER_W = _B_FLAT // _NW
_CH = 8
_NCHUNK = _B_PER_W // _CH
_ROWS_PER_TILE = _TOTAL_VT // _NS


def _merge_table(core_s, emb_hbm, train_hbm, tbl_hbm):
    row0 = core_s * _ROWS_PER_TILE

    @pl.when(core_s < _NUM_FIXED // _ROWS_PER_TILE)
    def _():
        pltpu.sync_copy(emb_hbm.at[pl.ds(row0, _ROWS_PER_TILE)],
                        tbl_hbm.at[pl.ds(row0, _ROWS_PER_TILE)])

    @pl.when(core_s >= _NUM_FIXED // _ROWS_PER_TILE)
    def _():
        pltpu.sync_copy(train_hbm.at[pl.ds(row0 - _NUM_FIXED, _ROWS_PER_TILE)],
                        tbl_hbm.at[pl.ds(row0, _ROWS_PER_TILE)])


def _gather(wid, tbl_hbm, idx_v, out_hbm, bufs, gsems, wsems):
    out0 = wid * _B_PER_W

    def start_gather(chunk, b):
        pltpu.async_copy(tbl_hbm.at[idx_v.at[chunk]], bufs[b], gsems[b])

    def wait_gather(b):
        pltpu.make_async_copy(tbl_hbm.at[pl.ds(0, _CH)], bufs[b], gsems[b]).wait()

    def start_write(chunk, b):
        pltpu.async_copy(bufs[b], out_hbm.at[pl.ds(out0 + chunk * _CH, _CH)],
                         wsems[b])

    def wait_write(chunk, b):
        pltpu.make_async_copy(bufs[b],
                              out_hbm.at[pl.ds(out0 + chunk * _CH, _CH)],
                              wsems[b]).wait()

    start_gather(0, 0)
    start_gather(1, 1)

    @pl.loop(0, _NCHUNK - 2, step=2)
    def _(k):
        for b in range(2):
            wait_gather(b)
            start_write(k + b, b)
        for b in range(2):
            wait_write(k + b, b)
            start_gather(k + 2 + b, b)

    k_last = _NCHUNK - 2
    for b in range(2):
        wait_gather(b)
        start_write(k_last + b, b)
    for b in range(2):
        wait_write(k_last + b, b)


def _sc_kernel_body(emb_hbm, train_hbm, idx_hbm, out_hbm,
                    tbl_a, tbl_b, idx_v, rows_v0, rows_v1,
                    gsem0, gsem1, wsem0, wsem1):
    c = lax.axis_index("c")
    s = lax.axis_index("s")
    wid = s * _NC + c

    @pl.when(c == 0)
    def _():
        _merge_table(s, emb_hbm, train_hbm, tbl_a)

    @pl.when(c == 1)
    def _():
        _merge_table(s, emb_hbm, train_hbm, tbl_b)

    pltpu.sync_copy(idx_hbm.at[pl.ds(wid * _NCHUNK, _NCHUNK)], idx_v)

    plsc.subcore_barrier()

    bufs = (rows_v0, rows_v1)
    gsems = (gsem0, gsem1)
    wsems = (wsem0, wsem1)

    @pl.when(c == 0)
    def _():
        _gather(wid, tbl_a, idx_v, out_hbm, bufs, gsems, wsems)

    @pl.when(c == 1)
    def _():
        _gather(wid, tbl_b, idx_v, out_hbm, bufs, gsems, wsems)


@jax.jit
def _run(indices_2d, embeddings_weight, trainable_weight):
    mesh = plsc.VectorSubcoreMesh(core_axis_name="c", subcore_axis_name="s")
    f = pl.kernel(
        _sc_kernel_body,
        out_type=jax.ShapeDtypeStruct((_B_FLAT, _TOKEN_DIM), jnp.float32),
        mesh=mesh,
        scratch_types=[
            pltpu.HBM((_TOTAL_VT, _TOKEN_DIM), jnp.float32),
            pltpu.HBM((_TOTAL_VT, _TOKEN_DIM), jnp.float32),
            pltpu.VMEM((_NCHUNK, _CH), jnp.int32),
            pltpu.VMEM((_CH, _TOKEN_DIM), jnp.float32),
            pltpu.VMEM((_CH, _TOKEN_DIM), jnp.float32),
            pltpu.SemaphoreType.DMA,
            pltpu.SemaphoreType.DMA,
            pltpu.SemaphoreType.DMA,
            pltpu.SemaphoreType.DMA,
        ],
    )
    return f(embeddings_weight, trainable_weight, indices_2d)


def kernel(indices, embeddings_weight, trainable_weight):
    idx_2d = indices.astype(jnp.int32).reshape(_B_FLAT // _CH, _CH)
    out = _run(idx_2d, embeddings_weight, trainable_weight)
    return out.reshape(_BATCH, _TOTAL_VT, _TOKEN_DIM)


# speedup vs baseline: 1.6922x; 1.6922x over previous
import jax
import jax.numpy as jnp
from jax import lax
from jax.experimental import pallas as pl
from jax.experimental.pallas import tpu as pltpu
from jax.experimental.pallas import tpu_sc as plsc

_NUM_FIXED = 256
_TOTAL_VT = 1024
_TOKEN_DIM = 4096
_BATCH = 16

_NC = 2
_NS = 16
_NW = _NC * _NS

_B_FLAT = _BATCH * _TOTAL_VT
_B_PER_W = _B_FLAT // _NW
_CH = 4
_NCHUNK = _B_PER_W // _CH
_NBUF = 7
_NFULL = _NCHUNK // _NBUF
_REM = _NCHUNK - _NFULL * _NBUF
_MROWS = _TOTAL_VT // _NW


def _merge_body(emb_hbm, train_hbm, tbl_hbm):
    c = lax.axis_index("c")
    s = lax.axis_index("s")
    wid = s * _NC + c
    row0 = wid * _MROWS

    @pl.when(row0 < _NUM_FIXED)
    def _():
        pltpu.sync_copy(emb_hbm.at[pl.ds(row0, _MROWS)],
                        tbl_hbm.at[pl.ds(row0, _MROWS)])

    @pl.when(row0 >= _NUM_FIXED)
    def _():
        pltpu.sync_copy(train_hbm.at[pl.ds(row0 - _NUM_FIXED, _MROWS)],
                        tbl_hbm.at[pl.ds(row0, _MROWS)])


def _gather_body(tbl_hbm, idx_hbm, out_hbm, idx_v, *rest):
    bufs = rest[:_NBUF]
    gsems = rest[_NBUF:2 * _NBUF]
    wsems = rest[2 * _NBUF:3 * _NBUF]

    c = lax.axis_index("c")
    s = lax.axis_index("s")
    wid = s * _NC + c
    out0 = wid * _B_PER_W

    pltpu.sync_copy(idx_hbm.at[pl.ds(wid * _NCHUNK, _NCHUNK)], idx_v)

    def start_gather(chunk, b):
        pltpu.async_copy(tbl_hbm.at[idx_v.at[chunk]], bufs[b], gsems[b])

    def wait_gather(b):
        pltpu.make_async_copy(tbl_hbm.at[pl.ds(0, _CH)], bufs[b], gsems[b]).wait()

    def start_write(chunk, b):
        pltpu.async_copy(bufs[b], out_hbm.at[pl.ds(out0 + chunk * _CH, _CH)],
                         wsems[b])

    def wait_write(chunk, b):
        pltpu.make_async_copy(bufs[b],
                              out_hbm.at[pl.ds(out0 + chunk * _CH, _CH)],
                              wsems[b]).wait()

    for b in range(_NBUF):
        start_gather(b, b)

    @pl.loop(0, (_NFULL - 1) * _NBUF, step=_NBUF)
    def _(base):
        for b in range(_NBUF):
            wait_gather(b)
            start_write(base + b, b)
        for b in range(_NBUF):
            wait_write(base + b, b)
            start_gather(base + _NBUF + b, b)

    last = (_NFULL - 1) * _NBUF
    for b in range(_NBUF):
        wait_gather(b)
        start_write(last + b, b)
    for r in range(_REM):
        wait_write(last + r, r)
        start_gather(_NFULL * _NBUF + r, r)
    for r in range(_REM):
        wait_gather(r)
        start_write(_NFULL * _NBUF + r, r)
    for b in range(_REM, _NBUF):
        wait_write(last + b, b)
    for r in range(_REM):
        wait_write(_NFULL * _NBUF + r, r)


@jax.jit
def _run(indices_2d, embeddings_weight, trainable_weight):
    mesh = plsc.VectorSubcoreMesh(core_axis_name="c", subcore_axis_name="s")
    merge = pl.kernel(
        _merge_body,
        out_type=jax.ShapeDtypeStruct((_TOTAL_VT, _TOKEN_DIM), jnp.float32),
        mesh=mesh,
    )
    merged = merge(embeddings_weight, trainable_weight)
    gather = pl.kernel(
        _gather_body,
        out_type=jax.ShapeDtypeStruct((_B_FLAT, _TOKEN_DIM), jnp.float32),
        mesh=mesh,
        scratch_types=(
            [pltpu.VMEM((_NCHUNK, _CH), jnp.int32)]
            + [pltpu.VMEM((_CH, _TOKEN_DIM), jnp.float32) for _ in range(_NBUF)]
            + [pltpu.SemaphoreType.DMA for _ in range(2 * _NBUF)]
        ),
    )
    return gather(merged, indices_2d)


def kernel(indices, embeddings_weight, trainable_weight):
    idx_2d = indices.astype(jnp.int32).reshape(_B_FLAT // _CH, _CH)
    out = _run(idx_2d, embeddings_weight, trainable_weight)
    return out.reshape(_BATCH, _TOTAL_VT, _TOKEN_DIM)


# speedup vs baseline: 5.4569x; 3.2248x over previous
import jax
import jax.numpy as jnp
from jax import lax
from jax.experimental import pallas as pl
from jax.experimental.pallas import tpu as pltpu
from jax.experimental.pallas import tpu_sc as plsc

_NUM_FIXED = 256
_TOTAL_VT = 1024
_TOKEN_DIM = 4096
_BATCH = 16

_NC = 2
_NS = 16
_NW = _NC * _NS

_B_FLAT = _BATCH * _TOTAL_VT
_B_PER_W = _B_FLAT // _NW
_CH = 4
_NCHUNK = _B_PER_W // _CH
_NBUF = 7
_NFULL = _NCHUNK // _NBUF
_REM = _NCHUNK - _NFULL * _NBUF
_MROWS = _TOTAL_VT // _NW


_MBLK = 128
_MGRID = _TOTAL_VT // _MBLK
_MFIX = _NUM_FIXED // _MBLK


def _merge_body(emb_ref, train_ref, out_ref):
    g = pl.program_id(0)

    @pl.when(g < _MFIX)
    def _():
        out_ref[...] = emb_ref[...]

    @pl.when(g >= _MFIX)
    def _():
        out_ref[...] = train_ref[...]


def _gather_body(tbl_hbm, idx_hbm, out_hbm, idx_v, *rest):
    bufs = rest[:_NBUF]
    gsems = rest[_NBUF:2 * _NBUF]
    wsems = rest[2 * _NBUF:3 * _NBUF]

    c = lax.axis_index("c")
    s = lax.axis_index("s")
    wid = s * _NC + c
    out0 = wid * _B_PER_W

    pltpu.sync_copy(idx_hbm.at[pl.ds(wid * _NCHUNK, _NCHUNK)], idx_v)

    def start_gather(chunk, b):
        pltpu.async_copy(tbl_hbm.at[idx_v.at[chunk]], bufs[b], gsems[b])

    def wait_gather(b):
        pltpu.make_async_copy(tbl_hbm.at[pl.ds(0, _CH)], bufs[b], gsems[b]).wait()

    def start_write(chunk, b):
        pltpu.async_copy(bufs[b], out_hbm.at[pl.ds(out0 + chunk * _CH, _CH)],
                         wsems[b])

    def wait_write(chunk, b):
        pltpu.make_async_copy(bufs[b],
                              out_hbm.at[pl.ds(out0 + chunk * _CH, _CH)],
                              wsems[b]).wait()

    for b in range(_NBUF):
        start_gather(b, b)

    @pl.loop(0, (_NFULL - 1) * _NBUF, step=_NBUF)
    def _(base):
        for b in range(_NBUF):
            wait_gather(b)
            start_write(base + b, b)
        for b in range(_NBUF):
            wait_write(base + b, b)
            start_gather(base + _NBUF + b, b)

    last = (_NFULL - 1) * _NBUF
    for b in range(_NBUF):
        wait_gather(b)
        start_write(last + b, b)
    for r in range(_REM):
        wait_write(last + r, r)
        start_gather(_NFULL * _NBUF + r, r)
    for r in range(_REM):
        wait_gather(r)
        start_write(_NFULL * _NBUF + r, r)
    for b in range(_REM, _NBUF):
        wait_write(last + b, b)
    for r in range(_REM):
        wait_write(_NFULL * _NBUF + r, r)


@jax.jit
def _run(indices_2d, embeddings_weight, trainable_weight):
    mesh = plsc.VectorSubcoreMesh(core_axis_name="c", subcore_axis_name="s")
    merged = pl.pallas_call(
        _merge_body,
        out_shape=jax.ShapeDtypeStruct((_TOTAL_VT, _TOKEN_DIM), jnp.float32),
        grid=(_MGRID,),
        in_specs=[
            pl.BlockSpec((_MBLK, _TOKEN_DIM),
                         lambda g: (jnp.minimum(g, _MFIX - 1), 0)),
            pl.BlockSpec((_MBLK, _TOKEN_DIM),
                         lambda g: (jnp.maximum(g, _MFIX) - _MFIX, 0)),
        ],
        out_specs=pl.BlockSpec((_MBLK, _TOKEN_DIM), lambda g: (g, 0)),
    )(embeddings_weight, trainable_weight)
    gather = pl.kernel(
        _gather_body,
        out_type=jax.ShapeDtypeStruct((_B_FLAT, _TOKEN_DIM), jnp.float32),
        mesh=mesh,
        scratch_types=(
            [pltpu.VMEM((_NCHUNK, _CH), jnp.int32)]
            + [pltpu.VMEM((_CH, _TOKEN_DIM), jnp.float32) for _ in range(_NBUF)]
            + [pltpu.SemaphoreType.DMA for _ in range(2 * _NBUF)]
        ),
    )
    return gather(merged, indices_2d)


def kernel(indices, embeddings_weight, trainable_weight):
    idx_2d = indices.astype(jnp.int32).reshape(_B_FLAT // _CH, _CH)
    out = _run(idx_2d, embeddings_weight, trainable_weight)
    return out.reshape(_BATCH, _TOTAL_VT, _TOKEN_DIM)


# speedup vs baseline: 5.4645x; 1.0014x over previous
import jax
import jax.numpy as jnp
from jax import lax
from jax.experimental import pallas as pl
from jax.experimental.pallas import tpu as pltpu
from jax.experimental.pallas import tpu_sc as plsc

_NUM_FIXED = 256
_TOTAL_VT = 1024
_TOKEN_DIM = 4096
_BATCH = 16

_NC = 2
_NS = 16
_NW = _NC * _NS

_B_FLAT = _BATCH * _TOTAL_VT
_B_PER_W = _B_FLAT // _NW
_CH = 8
_NCHUNK = _B_PER_W // _CH
_NBUF = 3
_NFULL = _NCHUNK // _NBUF
_REM = _NCHUNK - _NFULL * _NBUF
_MROWS = _TOTAL_VT // _NW


_MBLK = 128
_MGRID = _TOTAL_VT // _MBLK
_MFIX = _NUM_FIXED // _MBLK


def _merge_body(emb_ref, train_ref, out_ref):
    g = pl.program_id(0)

    @pl.when(g < _MFIX)
    def _():
        out_ref[...] = emb_ref[...]

    @pl.when(g >= _MFIX)
    def _():
        out_ref[...] = train_ref[...]


def _gather_body(tbl_hbm, idx_hbm, out_hbm, idx_v, *rest):
    bufs = rest[:_NBUF]
    gsems = rest[_NBUF:2 * _NBUF]
    wsems = rest[2 * _NBUF:3 * _NBUF]

    c = lax.axis_index("c")
    s = lax.axis_index("s")
    wid = s * _NC + c
    out0 = wid * _B_PER_W

    pltpu.sync_copy(idx_hbm.at[pl.ds(wid * _NCHUNK, _NCHUNK)], idx_v)

    def start_gather(chunk, b):
        pltpu.async_copy(tbl_hbm.at[idx_v.at[chunk]], bufs[b], gsems[b])

    def wait_gather(b):
        pltpu.make_async_copy(tbl_hbm.at[pl.ds(0, _CH)], bufs[b], gsems[b]).wait()

    def start_write(chunk, b):
        pltpu.async_copy(bufs[b], out_hbm.at[pl.ds(out0 + chunk * _CH, _CH)],
                         wsems[b])

    def wait_write(chunk, b):
        pltpu.make_async_copy(bufs[b],
                              out_hbm.at[pl.ds(out0 + chunk * _CH, _CH)],
                              wsems[b]).wait()

    for b in range(_NBUF):
        start_gather(b, b)

    @pl.loop(0, (_NFULL - 1) * _NBUF, step=_NBUF)
    def _(base):
        for b in range(_NBUF):
            wait_gather(b)
            start_write(base + b, b)
        for b in range(_NBUF):
            wait_write(base + b, b)
            start_gather(base + _NBUF + b, b)

    last = (_NFULL - 1) * _NBUF
    for b in range(_NBUF):
        wait_gather(b)
        start_write(last + b, b)
    for r in range(_REM):
        wait_write(last + r, r)
        start_gather(_NFULL * _NBUF + r, r)
    for r in range(_REM):
        wait_gather(r)
        start_write(_NFULL * _NBUF + r, r)
    for b in range(_REM, _NBUF):
        wait_write(last + b, b)
    for r in range(_REM):
        wait_write(_NFULL * _NBUF + r, r)


@jax.jit
def _run(indices_2d, embeddings_weight, trainable_weight):
    mesh = plsc.VectorSubcoreMesh(core_axis_name="c", subcore_axis_name="s")
    merged = pl.pallas_call(
        _merge_body,
        out_shape=jax.ShapeDtypeStruct((_TOTAL_VT, _TOKEN_DIM), jnp.float32),
        grid=(_MGRID,),
        in_specs=[
            pl.BlockSpec((_MBLK, _TOKEN_DIM),
                         lambda g: (jnp.minimum(g, _MFIX - 1), 0)),
            pl.BlockSpec((_MBLK, _TOKEN_DIM),
                         lambda g: (jnp.maximum(g, _MFIX) - _MFIX, 0)),
        ],
        out_specs=pl.BlockSpec((_MBLK, _TOKEN_DIM), lambda g: (g, 0)),
    )(embeddings_weight, trainable_weight)
    gather = pl.kernel(
        _gather_body,
        out_type=jax.ShapeDtypeStruct((_B_FLAT, _TOKEN_DIM), jnp.float32),
        mesh=mesh,
        scratch_types=(
            [pltpu.VMEM((_NCHUNK, _CH), jnp.int32)]
            + [pltpu.VMEM((_CH, _TOKEN_DIM), jnp.float32) for _ in range(_NBUF)]
            + [pltpu.SemaphoreType.DMA for _ in range(2 * _NBUF)]
        ),
    )
    return gather(merged, indices_2d)


def kernel(indices, embeddings_weight, trainable_weight):
    idx_2d = indices.astype(jnp.int32).reshape(_B_FLAT // _CH, _CH)
    out = _run(idx_2d, embeddings_weight, trainable_weight)
    return out.reshape(_BATCH, _TOTAL_VT, _TOKEN_DIM)
